# baseline (speedup 1.0000x reference)
import jax
import jax.numpy as jnp
from jax import lax
from jax.experimental import pallas as pl
from jax.experimental.pallas import tpu as pltpu
from jax.experimental.pallas import tpu_sc as plsc

BATCH = 16384
EMBED_DIM = 64
NUM_FEATURES = 13
NC = 2
NS = 16
NW = NC * NS
CHUNK = BATCH // NW
IDX_SUB = 128
NSUB = CHUNK // IDX_SUB
GP = IDX_SUB // 16


def _sc_body(uidx_hbm, midx_hbm, feat_hbm, utab_hbm, mtab_hbm, params_hbm,
             out_hbm, uidx_v, midx_v, urows_v, mrows_v, feat_v, w_v, tile_v,
             out_v, sem_u, sem_m):
    cid = lax.axis_index("c")
    sid = lax.axis_index("s")
    wid = sid * NC + cid
    base = wid * CHUNK

    pltpu.sync_copy(uidx_hbm.at[wid], uidx_v)
    pltpu.sync_copy(midx_hbm.at[wid], midx_v)
    copies = []
    for j in range(NSUB):
        copies.append(pltpu.async_copy(
            utab_hbm.at[uidx_v.at[j]], urows_v.at[pl.ds(j * IDX_SUB, IDX_SUB)],
            sem_u))
        copies.append(pltpu.async_copy(
            mtab_hbm.at[midx_v.at[j]], mrows_v.at[pl.ds(j * IDX_SUB, IDX_SUB)],
            sem_m))
    pltpu.sync_copy(feat_hbm.at[pl.ds(base, CHUNK)], feat_v)
    pltpu.sync_copy(params_hbm, w_v)

    lane = lax.iota(jnp.int32, 16)
    wc = [w_v[pl.ds(k * 16, 16)] for k in range(4)]
    wf = w_v[pl.ds(64, 16)]

    def group(g, carry):
        toff = (g % 2) * 256
        for h in range(2):
            ps = []
            for e in range(h * 8, h * 8 + 8):
                i = g * 16 + e
                terms = [feat_v[i, :] * wf]
                for k in range(4):
                    u = urows_v[i, pl.ds(k * 16, 16)]
                    m = mrows_v[i, pl.ds(k * 16, 16)]
                    terms.append((u * m) * wc[k])
                ps.append(((terms[0] + terms[1]) + (terms[2] + terms[3]))
                          + terms[4])
            for e in range(h * 8, h * 8 + 8):
                tile_v[pl.ds(toff + e * 16, 16)] = ps[e - h * 8]
        t0 = lane * 16 + toff
        cols = [plsc.load_gather(tile_v, [t0 + j]) for j in range(16)]
        while len(cols) > 1:
            cols = [cols[a] + cols[a + 1] for a in range(0, len(cols), 2)]
        out_v[pl.ds(g * 16, 16)] = cols[0]
        return carry

    for j in range(NSUB):
        copies[2 * j].wait()
        copies[2 * j + 1].wait()
        lax.fori_loop(j * GP, (j + 1) * GP, group, None, unroll=2)
    pltpu.sync_copy(out_v, out_hbm.at[pl.ds(base, CHUNK)])


@jax.jit
def _run(uidx3, midx3, feat16, user_table, movie_table, params):
    mesh = plsc.VectorSubcoreMesh(core_axis_name="c", subcore_axis_name="s",
                                  num_cores=NC, num_subcores=NS)
    f = pl.kernel(
        _sc_body,
        out_type=jax.ShapeDtypeStruct((BATCH,), jnp.float32),
        mesh=mesh,
        compiler_params=pltpu.CompilerParams(needs_layout_passes=False,
                                             use_tc_tiling_on_sc=False),
        scratch_types=[
            pltpu.VMEM((NSUB, IDX_SUB), jnp.int32),
            pltpu.VMEM((NSUB, IDX_SUB), jnp.int32),
            pltpu.VMEM((CHUNK, EMBED_DIM), jnp.float32),
            pltpu.VMEM((CHUNK, EMBED_DIM), jnp.float32),
            pltpu.VMEM((CHUNK, 16), jnp.float32),
            pltpu.VMEM((80,), jnp.float32),
            pltpu.VMEM((512,), jnp.float32),
            pltpu.VMEM((CHUNK,), jnp.float32),
            pltpu.SemaphoreType.DMA,
            pltpu.SemaphoreType.DMA,
        ],
    )
    return f(uidx3, midx3, feat16, user_table, movie_table, params)


def kernel(user_idx, movie_idx, features, user_table, movie_table, fc_w, fc_b):
    uidx3 = user_idx.astype(jnp.int32).reshape(NW, NSUB, IDX_SUB)
    midx3 = movie_idx.astype(jnp.int32).reshape(NW, NSUB, IDX_SUB)
    feat16 = jnp.concatenate(
        [features,
         jnp.ones((BATCH, 1), jnp.float32),
         jnp.zeros((BATCH, 2), jnp.float32)], axis=1)
    params = jnp.concatenate(
        [fc_w[0], fc_b, jnp.zeros((2,), jnp.float32)]).astype(jnp.float32)
    return _run(uidx3, midx3, feat16, user_table, movie_table, params)
